# Initial kernel scaffold; baseline (speedup 1.0000x reference)
import jax
import jax.numpy as jnp
from jax.experimental import pallas as pl


def kernel(token_indices, aa_table, group_table, aa_to_group, aa_properties, W1, b1, ln1_g, ln1_b, W2, b2, norm_g, norm_b):
    raise NotImplementedError("write your pallas kernel here")


# speedup vs baseline: 3.5728x; 3.5728x over previous
import functools
import math

import jax
import jax.numpy as jnp
from jax import lax
from jax.experimental import pallas as pl
from jax.experimental.pallas import tpu as pltpu
from jax.experimental.pallas import tpu_sc as plsc

_VOCAB = 22
_D_OUT = 56
_NC = 2
_NS = 16
_NW = _NC * _NS
_CHUNK = 128


def _table_body(aa_ref, gt_ref, g_ids_ref, props_ref, w1t_ref, b1_ref,
                ln1g_ref, ln1b_ref, w2t_ref, b2_ref, ng_ref, nb_ref, out_ref):
    f32 = jnp.float32
    aa = aa_ref[...]
    gids = g_ids_ref[...]
    onehot = (gids == lax.broadcasted_iota(jnp.int32, (_VOCAB, 5), 1)).astype(f32)
    group_emb = jnp.dot(onehot, gt_ref[...], preferred_element_type=f32)

    h = jnp.dot(props_ref[...], w1t_ref[...], preferred_element_type=f32)
    h = h + b1_ref[...]
    mean = jnp.mean(h, axis=1, keepdims=True)
    var = jnp.mean((h - mean) ** 2, axis=1, keepdims=True)
    h = (h - mean) * lax.rsqrt(var + 1e-5) * ln1g_ref[...] + ln1b_ref[...]
    h = 0.5 * h * (1.0 + lax.erf(h / math.sqrt(2.0)))
    prop_emb = jnp.dot(h, w2t_ref[...], preferred_element_type=f32) + b2_ref[...]

    comb = jnp.concatenate([aa, group_emb, prop_emb], axis=1)
    mean2 = jnp.mean(comb, axis=1, keepdims=True)
    var2 = jnp.mean((comb - mean2) ** 2, axis=1, keepdims=True)
    out_ref[...] = ((comb - mean2) * lax.rsqrt(var2 + 1e-5) * ng_ref[...]
                    + nb_ref[...])


def _build_table(aa_table, group_table, aa_to_group, aa_properties,
                 W1, b1, ln1_g, ln1_b, W2, b2, norm_g, norm_b):
    return pl.pallas_call(
        _table_body,
        out_shape=jax.ShapeDtypeStruct((_VOCAB, _D_OUT), jnp.float32),
    )(aa_table, group_table, aa_to_group.reshape(_VOCAB, 1).astype(jnp.int32),
      aa_properties, W1.T, b1.reshape(1, -1), ln1_g.reshape(1, -1),
      ln1_b.reshape(1, -1), W2.T, b2.reshape(1, -1), norm_g.reshape(1, -1),
      norm_b.reshape(1, -1))


def _gather_body(table_hbm, idx_hbm, out_hbm, idx_v, buf, sem):
    n_chunks_w = idx_v.shape[0]
    wid = lax.axis_index("s") * _NC + lax.axis_index("c")
    row0 = wid * n_chunks_w
    pltpu.sync_copy(idx_hbm.at[pl.ds(row0, n_chunks_w)], idx_v)

    def step(j, carry):
        pltpu.async_copy(table_hbm.at[idx_v.at[j]], buf, sem).wait()
        pltpu.sync_copy(buf, out_hbm.at[pl.ds((row0 + j) * _CHUNK, _CHUNK)])
        return carry

    lax.fori_loop(0, n_chunks_w, step, 0)


def _gather(table, idx_2d, n_tokens):
    n_chunks_w = idx_2d.shape[0] // _NW
    mesh = plsc.VectorSubcoreMesh(core_axis_name="c", subcore_axis_name="s")
    return pl.kernel(
        _gather_body,
        out_type=jax.ShapeDtypeStruct((n_tokens, _D_OUT), jnp.float32),
        mesh=mesh,
        scratch_types=[
            pltpu.VMEM((n_chunks_w, _CHUNK), jnp.int32),
            pltpu.VMEM((_CHUNK, _D_OUT), jnp.float32),
            pltpu.SemaphoreType.DMA,
        ],
        compiler_params=pltpu.CompilerParams(use_tc_tiling_on_sc=False),
    )(table, idx_2d)


def kernel(token_indices, aa_table, group_table, aa_to_group, aa_properties,
           W1, b1, ln1_g, ln1_b, W2, b2, norm_g, norm_b):
    n_rows, n_cols = token_indices.shape
    n_tokens = n_rows * n_cols
    table = _build_table(aa_table, group_table, aa_to_group, aa_properties,
                         W1, b1, ln1_g, ln1_b, W2, b2, norm_g, norm_b)
    idx_2d = token_indices.reshape(n_tokens // _CHUNK, _CHUNK).astype(jnp.int32)
    out = _gather(table, idx_2d, n_tokens)
    return out.reshape(n_rows, n_cols, _D_OUT)
